# baseline (speedup 1.0000x reference)
import functools

import jax
import jax.numpy as jnp
from jax import lax
from jax.experimental import pallas as pl
from jax.experimental.pallas import tpu as pltpu
from jax.experimental.pallas import tpu_sc as plsc

_EPS = 1e-5


def _logits_body(x_ref, w1_ref, b1_ref, g_ref, be_ref, rm_ref, rv_ref, w2_ref,
                 b2_ref, out_ref):
    x = x_ref[...]
    w1 = w1_ref[...]
    h = jnp.dot(x[:, 0:256], w1[0:256, :], preferred_element_type=jnp.float32)
    for k0 in range(256, x.shape[1], 256):
        h = h + jnp.dot(x[:, k0:k0 + 256], w1[k0:k0 + 256, :],
                        preferred_element_type=jnp.float32)
    h = h + b1_ref[...]
    h = (h - rm_ref[...]) / jnp.sqrt(rv_ref[...] + _EPS) * g_ref[...] + be_ref[...]
    h = jnp.maximum(h, 0.0)
    out_ref[...] = jnp.dot(h, w2_ref[...],
                           preferred_element_type=jnp.float32) + b2_ref[...]


def _compute_logits(x, W1t, b1, gamma, beta, run_mean, run_var, W2t, b2,
                    n_rows, row_off):
    d = x.shape[1]
    hidden = W1t.shape[1]
    n_e = W2t.shape[1]
    t = 512
    off_tiles = row_off // t
    vec = pl.BlockSpec((1, hidden), lambda i: (0, 0))
    return pl.pallas_call(
        _logits_body,
        grid=(n_rows // t,),
        in_specs=[
            pl.BlockSpec((t, d), lambda i: (i + off_tiles, 0)),
            pl.BlockSpec((d, hidden), lambda i: (0, 0)),
            vec, vec, vec, vec, vec,
            pl.BlockSpec((hidden, n_e), lambda i: (0, 0)),
            pl.BlockSpec((1, n_e), lambda i: (0, 0)),
        ],
        out_specs=pl.BlockSpec((t, n_e), lambda i: (i, 0)),
        out_shape=jax.ShapeDtypeStruct((n_rows, n_e), jnp.float32),
    )(x, W1t, b1, gamma, beta, run_mean, run_var, W2t, b2)


def _make_router(n, n_e):
    info = plsc.get_sparse_core_info()
    nw = info.num_cores * info.num_subcores
    t_w = n // nw
    n_groups = t_w // 16
    mesh = plsc.VectorSubcoreMesh(core_axis_name="c", subcore_axis_name="s")

    @functools.partial(
        pl.kernel, mesh=mesh,
        compiler_params=pltpu.CompilerParams(needs_layout_passes=False),
        out_type=[jax.ShapeDtypeStruct((n * n_e,), jnp.float32),
                  jax.ShapeDtypeStruct((n * 2,), jnp.int32)],
        scratch_types=[pltpu.VMEM((t_w * n_e,), jnp.float32),
                       pltpu.VMEM((t_w * n_e,), jnp.float32),
                       pltpu.VMEM((t_w * 2,), jnp.int32)],
    )
    def route(logits_hbm, gates_hbm, idx_hbm, lg_v, gt_v, ix_v):
        wid = lax.axis_index("s") * info.num_cores + lax.axis_index("c")
        base = wid * t_w
        pltpu.sync_copy(logits_hbm.at[pl.ds(base * n_e, t_w * n_e)], lg_v)

        def group(g, carry):
            tvec = lax.iota(jnp.int32, 16) + g * 16
            neg = jnp.full((16,), -3.4e38, jnp.float32)
            zero_i = jnp.zeros((16,), jnp.int32)
            max1, max2 = neg, neg
            idx1, idx2 = zero_i, zero_i
            row0 = tvec * n_e

            def scan_expert(e, carry):
                max1, idx1, max2, idx2 = carry
                evec = jnp.full((16,), 0, jnp.int32) + e
                col = plsc.load_gather(lg_v, [row0 + e])
                beats1 = col > max1
                beats2 = col > max2
                max2 = jnp.where(beats1, max1, jnp.where(beats2, col, max2))
                idx2 = jnp.where(beats1, idx1, jnp.where(beats2, evec, idx2))
                max1 = jnp.where(beats1, col, max1)
                idx1 = jnp.where(beats1, evec, idx1)
                return max1, idx1, max2, idx2

            max1, idx1, max2, idx2 = lax.fori_loop(
                0, n_e, scan_expert, (max1, idx1, max2, idx2))
            r = jnp.exp(max2 - max1)
            den = 1.0 + r
            g1 = 1.0 / den
            g2 = r / den
            fzero = jnp.zeros((16,), jnp.float32)

            def write_expert(e, carry):
                evec = jnp.full((16,), 0, jnp.int32) + e
                colg = jnp.where(idx1 == evec, g1,
                                 jnp.where(idx2 == evec, g2, fzero))
                plsc.store_scatter(gt_v, [row0 + e], colg)
                return carry

            lax.fori_loop(0, n_e, write_expert, 0)
            plsc.store_scatter(ix_v, [tvec * 2], idx1)
            plsc.store_scatter(ix_v, [tvec * 2 + 1], idx2)
            return carry

        lax.fori_loop(0, n_groups, group, 0)
        pltpu.sync_copy(gt_v, gates_hbm.at[pl.ds(base * n_e, t_w * n_e)])
        pltpu.sync_copy(ix_v, idx_hbm.at[pl.ds(base * 2, t_w * 2)])

    return route


_N_CHUNKS = 2


@jax.jit
def kernel(x, W1, b1, gamma, beta, run_mean, run_var, W2, b2):
    n = x.shape[0]
    n_e = W2.shape[0]
    w1t = W1.T
    w2t = W2.T
    args = (w1t, b1[None, :], gamma[None, :], beta[None, :], run_mean[None, :],
            run_var[None, :], w2t, b2[None, :])
    n_c = n // _N_CHUNKS
    router = _make_router(n_c, n_e)
    gates_parts = []
    idx_parts = []
    for c in range(_N_CHUNKS):
        logits = _compute_logits(x, *args, n_rows=n_c, row_off=c * n_c)
        g_flat, i_flat = router(logits.reshape(-1))
        gates_parts.append(g_flat.reshape(n_c, n_e))
        idx_parts.append(i_flat.reshape(n_c, 2))
    return (jnp.concatenate(gates_parts, axis=0),
            jnp.concatenate(idx_parts, axis=0))


# speedup vs baseline: 1.0611x; 1.0611x over previous
import functools

import jax
import jax.numpy as jnp
from jax import lax
from jax.experimental import pallas as pl
from jax.experimental.pallas import tpu as pltpu
from jax.experimental.pallas import tpu_sc as plsc

_EPS = 1e-5


def _logits(x, w1, b1, g, be, rm, rv, w2, b2):
    h = jnp.dot(x[:, 0:256], w1[0:256, :], preferred_element_type=jnp.float32)
    for k0 in range(256, x.shape[1], 256):
        h = h + jnp.dot(x[:, k0:k0 + 256], w1[k0:k0 + 256, :],
                        preferred_element_type=jnp.float32)
    h = h + b1
    h = (h - rm) / jnp.sqrt(rv + _EPS) * g + be
    h = jnp.maximum(h, 0.0)
    return jnp.dot(h, w2, preferred_element_type=jnp.float32) + b2


def _logits_body(x_ref, w1_ref, b1_ref, g_ref, be_ref, rm_ref, rv_ref, w2_ref,
                 b2_ref, out_ref):
    out_ref[...] = _logits(x_ref[...], w1_ref[...], b1_ref[...], g_ref[...],
                           be_ref[...], rm_ref[...], rv_ref[...], w2_ref[...],
                           b2_ref[...])


def _fused_body(x_ref, w1_ref, b1_ref, g_ref, be_ref, rm_ref, rv_ref, w2_ref,
                b2_ref, gates_ref, idx_ref):
    logits = _logits(x_ref[...], w1_ref[...], b1_ref[...], g_ref[...],
                     be_ref[...], rm_ref[...], rv_ref[...], w2_ref[...],
                     b2_ref[...])
    n_e = logits.shape[1]
    iota_e = jax.lax.broadcasted_iota(jnp.int32, logits.shape, 1)
    max1 = jnp.max(logits, axis=1, keepdims=True)
    idx1 = jnp.min(jnp.where(logits == max1, iota_e, n_e), axis=1, keepdims=True)
    masked = jnp.where(iota_e == idx1, -1e30, logits)
    max2 = jnp.max(masked, axis=1, keepdims=True)
    idx2 = jnp.min(jnp.where(masked == max2, iota_e, n_e), axis=1, keepdims=True)
    r = jnp.exp(max2 - max1)
    den = 1.0 + r
    g1 = 1.0 / den
    g2 = r / den
    gates_ref[...] = (jnp.where(iota_e == idx1, g1, 0.0)
                      + jnp.where(iota_e == idx2, g2, 0.0))
    idx_ref[...] = jnp.concatenate([idx1, idx2], axis=1).astype(jnp.int32)


def _compute_fused(x, W1t, b1, gamma, beta, run_mean, run_var, W2t, b2,
                   n_rows, row_off):
    d = x.shape[1]
    hidden = W1t.shape[1]
    n_e = W2t.shape[1]
    t = 512
    off_tiles = row_off // t
    vec = pl.BlockSpec((1, hidden), lambda i: (0, 0))
    return pl.pallas_call(
        _fused_body,
        grid=(n_rows // t,),
        in_specs=[
            pl.BlockSpec((t, d), lambda i: (i + off_tiles, 0)),
            pl.BlockSpec((d, hidden), lambda i: (0, 0)),
            vec, vec, vec, vec, vec,
            pl.BlockSpec((hidden, n_e), lambda i: (0, 0)),
            pl.BlockSpec((1, n_e), lambda i: (0, 0)),
        ],
        out_specs=[
            pl.BlockSpec((t, n_e), lambda i: (i, 0)),
            pl.BlockSpec((t, 2), lambda i: (i, 0)),
        ],
        out_shape=[
            jax.ShapeDtypeStruct((n_rows, n_e), jnp.float32),
            jax.ShapeDtypeStruct((n_rows, 2), jnp.int32),
        ],
    )(x, W1t, b1, gamma, beta, run_mean, run_var, W2t, b2)


def _compute_logits(x, W1t, b1, gamma, beta, run_mean, run_var, W2t, b2,
                    n_rows, row_off):
    d = x.shape[1]
    hidden = W1t.shape[1]
    n_e = W2t.shape[1]
    t = 512
    off_tiles = row_off // t
    vec = pl.BlockSpec((1, hidden), lambda i: (0, 0))
    return pl.pallas_call(
        _logits_body,
        grid=(n_rows // t,),
        in_specs=[
            pl.BlockSpec((t, d), lambda i: (i + off_tiles, 0)),
            pl.BlockSpec((d, hidden), lambda i: (0, 0)),
            vec, vec, vec, vec, vec,
            pl.BlockSpec((hidden, n_e), lambda i: (0, 0)),
            pl.BlockSpec((1, n_e), lambda i: (0, 0)),
        ],
        out_specs=pl.BlockSpec((t, n_e), lambda i: (i, 0)),
        out_shape=jax.ShapeDtypeStruct((n_rows, n_e), jnp.float32),
    )(x, W1t, b1, gamma, beta, run_mean, run_var, W2t, b2)


def _make_router(n, n_e):
    info = plsc.get_sparse_core_info()
    nw = info.num_cores * info.num_subcores
    t_w = n // nw
    n_groups = t_w // 16
    mesh = plsc.VectorSubcoreMesh(core_axis_name="c", subcore_axis_name="s")

    @functools.partial(
        pl.kernel, mesh=mesh,
        compiler_params=pltpu.CompilerParams(needs_layout_passes=False),
        out_type=[jax.ShapeDtypeStruct((n * n_e,), jnp.float32),
                  jax.ShapeDtypeStruct((n * 2,), jnp.int32)],
        scratch_types=[pltpu.VMEM((t_w * n_e,), jnp.float32),
                       pltpu.VMEM((t_w * n_e,), jnp.float32),
                       pltpu.VMEM((t_w * 2,), jnp.int32)],
    )
    def route(logits_hbm, gates_hbm, idx_hbm, lg_v, gt_v, ix_v):
        wid = lax.axis_index("s") * info.num_cores + lax.axis_index("c")
        base = wid * t_w
        pltpu.sync_copy(logits_hbm.at[pl.ds(base * n_e, t_w * n_e)], lg_v)

        def group(g, carry):
            tvec = lax.iota(jnp.int32, 16) + g * 16
            neg = jnp.full((16,), -3.4e38, jnp.float32)
            zero_i = jnp.zeros((16,), jnp.int32)
            max1, max2 = neg, neg
            idx1, idx2 = zero_i, zero_i
            row0 = tvec * n_e

            def scan_expert(e, carry):
                max1, idx1, max2, idx2 = carry
                evec = jnp.full((16,), 0, jnp.int32) + e
                col = plsc.load_gather(lg_v, [row0 + e])
                beats1 = col > max1
                beats2 = col > max2
                max2 = jnp.where(beats1, max1, jnp.where(beats2, col, max2))
                idx2 = jnp.where(beats1, idx1, jnp.where(beats2, evec, idx2))
                max1 = jnp.where(beats1, col, max1)
                idx1 = jnp.where(beats1, evec, idx1)
                return max1, idx1, max2, idx2

            max1, idx1, max2, idx2 = lax.fori_loop(
                0, n_e, scan_expert, (max1, idx1, max2, idx2))
            r = jnp.exp(max2 - max1)
            den = 1.0 + r
            g1 = 1.0 / den
            g2 = r / den
            fzero = jnp.zeros((16,), jnp.float32)

            def write_expert(e, carry):
                evec = jnp.full((16,), 0, jnp.int32) + e
                colg = jnp.where(idx1 == evec, g1,
                                 jnp.where(idx2 == evec, g2, fzero))
                plsc.store_scatter(gt_v, [row0 + e], colg)
                return carry

            lax.fori_loop(0, n_e, write_expert, 0)
            plsc.store_scatter(ix_v, [tvec * 2], idx1)
            plsc.store_scatter(ix_v, [tvec * 2 + 1], idx2)
            return carry

        lax.fori_loop(0, n_groups, group, 0)
        pltpu.sync_copy(gt_v, gates_hbm.at[pl.ds(base * n_e, t_w * n_e)])
        pltpu.sync_copy(ix_v, idx_hbm.at[pl.ds(base * 2, t_w * 2)])

    return route


_N_CHUNKS = 2


@jax.jit
def kernel(x, W1, b1, gamma, beta, run_mean, run_var, W2, b2):
    n = x.shape[0]
    n_e = W2.shape[0]
    w1t = W1.T
    w2t = W2.T
    args = (w1t, b1[None, :], gamma[None, :], beta[None, :], run_mean[None, :],
            run_var[None, :], w2t, b2[None, :])
    n_c = n // _N_CHUNKS
    router = _make_router(n_c, n_e)
    gates_parts = []
    idx_parts = []
    for c in range(_N_CHUNKS):
        if c < _N_CHUNKS - 1:
            logits = _compute_logits(x, *args, n_rows=n_c, row_off=c * n_c)
            g_flat, i_flat = router(logits.reshape(-1))
            gates_parts.append(g_flat.reshape(n_c, n_e))
            idx_parts.append(i_flat.reshape(n_c, 2))
        else:
            g_c, i_c = _compute_fused(x, *args, n_rows=n_c, row_off=c * n_c)
            gates_parts.append(g_c)
            idx_parts.append(i_c)
    return (jnp.concatenate(gates_parts, axis=0),
            jnp.concatenate(idx_parts, axis=0))


# speedup vs baseline: 1.1245x; 1.0597x over previous
import functools

import jax
import jax.numpy as jnp
from jax import lax
from jax.experimental import pallas as pl
from jax.experimental.pallas import tpu as pltpu
from jax.experimental.pallas import tpu_sc as plsc

_EPS = 1e-5


def _logits(x, w1, b1, g, be, rm, rv, w2, b2):
    h = jnp.dot(x[:, 0:256], w1[0:256, :], preferred_element_type=jnp.float32)
    for k0 in range(256, x.shape[1], 256):
        h = h + jnp.dot(x[:, k0:k0 + 256], w1[k0:k0 + 256, :],
                        preferred_element_type=jnp.float32)
    h = h + b1
    h = (h - rm) / jnp.sqrt(rv + _EPS) * g + be
    h = jnp.maximum(h, 0.0)
    return jnp.dot(h, w2, preferred_element_type=jnp.float32) + b2


def _logits_body(x_ref, w1_ref, b1_ref, g_ref, be_ref, rm_ref, rv_ref, w2_ref,
                 b2_ref, out_ref):
    out_ref[...] = _logits(x_ref[...], w1_ref[...], b1_ref[...], g_ref[...],
                           be_ref[...], rm_ref[...], rv_ref[...], w2_ref[...],
                           b2_ref[...])


def _fused_body(x_ref, w1_ref, b1_ref, g_ref, be_ref, rm_ref, rv_ref, w2_ref,
                b2_ref, gates_ref, idx_ref):
    logits = _logits(x_ref[...], w1_ref[...], b1_ref[...], g_ref[...],
                     be_ref[...], rm_ref[...], rv_ref[...], w2_ref[...],
                     b2_ref[...])
    n_e = logits.shape[1]
    iota_e = jax.lax.broadcasted_iota(jnp.int32, logits.shape, 1)
    max1 = jnp.max(logits, axis=1, keepdims=True)
    idx1 = jnp.min(jnp.where(logits == max1, iota_e, n_e), axis=1, keepdims=True)
    masked = jnp.where(iota_e == idx1, -1e30, logits)
    max2 = jnp.max(masked, axis=1, keepdims=True)
    idx2 = jnp.min(jnp.where(masked == max2, iota_e, n_e), axis=1, keepdims=True)
    r = jnp.exp(max2 - max1)
    den = 1.0 + r
    g1 = 1.0 / den
    g2 = r / den
    gates_ref[...] = (jnp.where(iota_e == idx1, g1, 0.0)
                      + jnp.where(iota_e == idx2, g2, 0.0))
    idx_ref[...] = jnp.concatenate([idx1, idx2], axis=1).astype(jnp.int32)


def _compute_fused(x, W1t, b1, gamma, beta, run_mean, run_var, W2t, b2,
                   n_rows, row_off):
    d = x.shape[1]
    hidden = W1t.shape[1]
    n_e = W2t.shape[1]
    t = 512
    off_tiles = row_off // t
    vec = pl.BlockSpec((1, hidden), lambda i: (0, 0))
    return pl.pallas_call(
        _fused_body,
        grid=(n_rows // t,),
        in_specs=[
            pl.BlockSpec((t, d), lambda i: (i + off_tiles, 0)),
            pl.BlockSpec((d, hidden), lambda i: (0, 0)),
            vec, vec, vec, vec, vec,
            pl.BlockSpec((hidden, n_e), lambda i: (0, 0)),
            pl.BlockSpec((1, n_e), lambda i: (0, 0)),
        ],
        out_specs=[
            pl.BlockSpec((t, n_e), lambda i: (i, 0)),
            pl.BlockSpec((t, 2), lambda i: (i, 0)),
        ],
        out_shape=[
            jax.ShapeDtypeStruct((n_rows, n_e), jnp.float32),
            jax.ShapeDtypeStruct((n_rows, 2), jnp.int32),
        ],
    )(x, W1t, b1, gamma, beta, run_mean, run_var, W2t, b2)


def _compute_logits(x, W1t, b1, gamma, beta, run_mean, run_var, W2t, b2,
                    n_rows, row_off):
    d = x.shape[1]
    hidden = W1t.shape[1]
    n_e = W2t.shape[1]
    t = 512
    off_tiles = row_off // t
    vec = pl.BlockSpec((1, hidden), lambda i: (0, 0))
    return pl.pallas_call(
        _logits_body,
        grid=(n_rows // t,),
        in_specs=[
            pl.BlockSpec((t, d), lambda i: (i + off_tiles, 0)),
            pl.BlockSpec((d, hidden), lambda i: (0, 0)),
            vec, vec, vec, vec, vec,
            pl.BlockSpec((hidden, n_e), lambda i: (0, 0)),
            pl.BlockSpec((1, n_e), lambda i: (0, 0)),
        ],
        out_specs=pl.BlockSpec((t, n_e), lambda i: (i, 0)),
        out_shape=jax.ShapeDtypeStruct((n_rows, n_e), jnp.float32),
    )(x, W1t, b1, gamma, beta, run_mean, run_var, W2t, b2)


def _make_router(n, n_e):
    info = plsc.get_sparse_core_info()
    nw = info.num_cores * info.num_subcores
    t_w = n // nw
    n_groups = t_w // 16
    mesh = plsc.VectorSubcoreMesh(core_axis_name="c", subcore_axis_name="s")

    @functools.partial(
        pl.kernel, mesh=mesh,
        compiler_params=pltpu.CompilerParams(needs_layout_passes=False),
        out_type=[jax.ShapeDtypeStruct((n, n_e), jnp.float32),
                  jax.ShapeDtypeStruct((n, 2), jnp.int32)],
        scratch_types=[pltpu.VMEM((t_w, n_e), jnp.float32),
                       pltpu.VMEM((t_w, n_e), jnp.float32),
                       pltpu.VMEM((t_w, 2), jnp.int32)],
    )
    def route(logits_hbm, gates_hbm, idx_hbm, lg_v, gt_v, ix_v):
        wid = lax.axis_index("s") * info.num_cores + lax.axis_index("c")
        base = wid * t_w
        pltpu.sync_copy(logits_hbm.at[pl.ds(base, t_w)], lg_v)

        def group(g, carry):
            tvec = lax.iota(jnp.int32, 16) + g * 16
            neg = jnp.full((16,), -3.4e38, jnp.float32)
            zero_i = jnp.zeros((16,), jnp.int32)
            max1, max2 = neg, neg
            idx1, idx2 = zero_i, zero_i
            def scan_expert(e, carry):
                max1, idx1, max2, idx2 = carry
                evec = jnp.full((16,), 0, jnp.int32) + e
                col = plsc.load_gather(lg_v, [tvec, evec])
                beats1 = col > max1
                beats2 = col > max2
                max2 = jnp.where(beats1, max1, jnp.where(beats2, col, max2))
                idx2 = jnp.where(beats1, idx1, jnp.where(beats2, evec, idx2))
                max1 = jnp.where(beats1, col, max1)
                idx1 = jnp.where(beats1, evec, idx1)
                return max1, idx1, max2, idx2

            max1, idx1, max2, idx2 = lax.fori_loop(
                0, n_e, scan_expert, (max1, idx1, max2, idx2))
            r = jnp.exp(max2 - max1)
            den = 1.0 + r
            g1 = 1.0 / den
            g2 = r / den
            fzero = jnp.zeros((16,), jnp.float32)

            def write_expert(e, carry):
                evec = jnp.full((16,), 0, jnp.int32) + e
                colg = jnp.where(idx1 == evec, g1,
                                 jnp.where(idx2 == evec, g2, fzero))
                plsc.store_scatter(gt_v, [tvec, evec], colg)
                return carry

            lax.fori_loop(0, n_e, write_expert, 0)
            plsc.store_scatter(ix_v, [tvec, zero_i], idx1)
            plsc.store_scatter(ix_v, [tvec, zero_i + 1], idx2)
            return carry

        lax.fori_loop(0, n_groups, group, 0)
        pltpu.sync_copy(gt_v, gates_hbm.at[pl.ds(base, t_w)])
        pltpu.sync_copy(ix_v, idx_hbm.at[pl.ds(base, t_w)])

    return route


_N_CHUNKS = 2


@jax.jit
def kernel(x, W1, b1, gamma, beta, run_mean, run_var, W2, b2):
    n = x.shape[0]
    n_e = W2.shape[0]
    w1t = W1.T
    w2t = W2.T
    args = (w1t, b1[None, :], gamma[None, :], beta[None, :], run_mean[None, :],
            run_var[None, :], w2t, b2[None, :])
    n_c = n // _N_CHUNKS
    router = _make_router(n_c, n_e)
    gates_parts = []
    idx_parts = []
    for c in range(_N_CHUNKS):
        if c < _N_CHUNKS - 1:
            logits = _compute_logits(x, *args, n_rows=n_c, row_off=c * n_c)
            g_c, i_c = router(logits)
            gates_parts.append(g_c)
            idx_parts.append(i_c)
        else:
            g_c, i_c = _compute_fused(x, *args, n_rows=n_c, row_off=c * n_c)
            gates_parts.append(g_c)
            idx_parts.append(i_c)
    return (jnp.concatenate(gates_parts, axis=0),
            jnp.concatenate(idx_parts, axis=0))
